# Initial kernel scaffold; baseline (speedup 1.0000x reference)
import jax
import jax.numpy as jnp
from jax.experimental import pallas as pl


def kernel(alpha, beta, gamma, alpha_idx, beta_idx, gamma_idx):
    raise NotImplementedError("write your pallas kernel here")


# speedup vs baseline: 1.1051x; 1.1051x over previous
import functools

import jax
import jax.numpy as jnp
from jax import lax
from jax.experimental import pallas as pl
from jax.experimental.pallas import tpu as pltpu
from jax.experimental.pallas import tpu_sc as plsc

_VERY_SMALL = 1e-8


def _not_gamma(gamma):
    B, N = gamma.shape
    blk = 64

    def body(g_ref, o_ref):
        g = g_ref[...]
        o_ref[...] = jnp.log(jnp.maximum(1.0 - jnp.exp(g), _VERY_SMALL))

    return pl.pallas_call(
        body,
        grid=(B // blk,),
        in_specs=[pl.BlockSpec((blk, N), lambda i: (i, 0))],
        out_specs=pl.BlockSpec((blk, N), lambda i: (i, 0)),
        out_shape=jax.ShapeDtypeStruct((B, N), jnp.float32),
    )(gamma)


def _pack_idx(idx):
    r = idx.astype(jnp.int32).reshape(-1, 2, 16)
    return (r[:, 0, :] | (r[:, 1, :] << 16)).reshape(-1)


@functools.cache
def _sc_gather(B, N, K):
    NC, NS = 2, 16
    NW = NC * NS
    RPT = B // NW
    OCH = 8192
    GRP = OCH // 32
    KP = K // 2
    assert K == 4 * OCH and B % NW == 0 and K % 32 == 0

    mesh = plsc.VectorSubcoreMesh(core_axis_name="c", subcore_axis_name="s")

    @functools.partial(
        pl.kernel,
        mesh=mesh,
        compiler_params=pltpu.CompilerParams(needs_layout_passes=False),
        out_type=jax.ShapeDtypeStruct((B, K), jnp.float32),
        scratch_types=[
            pltpu.VMEM((KP,), jnp.int32),
            pltpu.VMEM((KP,), jnp.int32),
            pltpu.VMEM((KP,), jnp.int32),
            pltpu.VMEM((N,), jnp.float32),
            pltpu.VMEM((N,), jnp.float32),
            pltpu.VMEM((N,), jnp.float32),
            pltpu.VMEM((OCH,), jnp.float32),
            pltpu.VMEM((OCH,), jnp.float32),
            pltpu.SemaphoreType.DMA,
            pltpu.SemaphoreType.DMA,
            pltpu.SemaphoreType.DMA,
        ],
    )
    def sc(a_hbm, b_hbm, g_hbm, pai_hbm, pbi_hbm, pgi_hbm, out_hbm,
           pai, pbi, pgi, arow, brow, grow, ob0, ob1,
           sem_in, sem_o0, sem_o1):
        wid = lax.axis_index("s") * NC + lax.axis_index("c")
        pltpu.sync_copy(pai_hbm, pai)
        pltpu.sync_copy(pbi_hbm, pbi)
        pltpu.sync_copy(pgi_hbm, pgi)
        row0 = wid * RPT
        m16 = jnp.int32(0xFFFF)

        def do_chunk(c, ob):
            wbase = c * (OCH // 2)

            def grp(g, carry):
                w = wbase + g * 16
                wa = pai[pl.ds(w, 16)]
                wb = pbi[pl.ds(w, 16)]
                wg = pgi[pl.ds(w, 16)]
                alo = plsc.load_gather(arow, [lax.bitwise_and(wa, m16)])
                ahi = plsc.load_gather(arow, [lax.shift_right_logical(wa, 16)])
                blo = plsc.load_gather(brow, [lax.bitwise_and(wb, m16)])
                bhi = plsc.load_gather(brow, [lax.shift_right_logical(wb, 16)])
                glo = plsc.load_gather(grow, [lax.bitwise_and(wg, m16)])
                ghi = plsc.load_gather(grow, [lax.shift_right_logical(wg, 16)])
                o = g * 32
                ob[pl.ds(o, 16)] = jnp.maximum(alo + blo - glo, 0.0)
                ob[pl.ds(o + 16, 16)] = jnp.maximum(ahi + bhi - ghi, 0.0)
                return carry

            lax.fori_loop(0, GRP, grp, 0)

        def row(r, carry):
            b = row0 + r
            ca = pltpu.async_copy(a_hbm.at[b], arow, sem_in)
            cb = pltpu.async_copy(b_hbm.at[b], brow, sem_in)
            cg = pltpu.async_copy(g_hbm.at[b], grow, sem_in)
            ca.wait()
            cb.wait()
            cg.wait()
            do_chunk(0, ob0)
            o0 = pltpu.async_copy(ob0, out_hbm.at[b, pl.ds(0 * OCH, OCH)], sem_o0)
            do_chunk(1, ob1)
            o1 = pltpu.async_copy(ob1, out_hbm.at[b, pl.ds(1 * OCH, OCH)], sem_o1)
            o0.wait()
            do_chunk(2, ob0)
            o2 = pltpu.async_copy(ob0, out_hbm.at[b, pl.ds(2 * OCH, OCH)], sem_o0)
            o1.wait()
            do_chunk(3, ob1)
            o3 = pltpu.async_copy(ob1, out_hbm.at[b, pl.ds(3 * OCH, OCH)], sem_o1)
            o2.wait()
            o3.wait()
            return carry

        lax.fori_loop(0, RPT, row, 0)

    return sc


def kernel(alpha, beta, gamma, alpha_idx, beta_idx, gamma_idx):
    B, N = alpha.shape
    K = alpha_idx.shape[0]
    ng = _not_gamma(gamma)
    pai = _pack_idx(alpha_idx)
    pbi = _pack_idx(beta_idx)
    pgi = _pack_idx(gamma_idx)
    return _sc_gather(B, N, K)(alpha, beta, ng, pai, pbi, pgi)


# speedup vs baseline: 1.9380x; 1.7536x over previous
import functools

import jax
import jax.numpy as jnp
from jax import lax
from jax.experimental import pallas as pl
from jax.experimental.pallas import tpu as pltpu
from jax.experimental.pallas import tpu_sc as plsc

_VERY_SMALL = 1e-8


def _not_gamma(gamma):
    B, N = gamma.shape
    blk = 64

    def body(g_ref, o_ref):
        g = g_ref[...]
        o_ref[...] = jnp.log(jnp.maximum(1.0 - jnp.exp(g), _VERY_SMALL))

    return pl.pallas_call(
        body,
        grid=(B // blk,),
        in_specs=[pl.BlockSpec((blk, N), lambda i: (i, 0))],
        out_specs=pl.BlockSpec((blk, N), lambda i: (i, 0)),
        out_shape=jax.ShapeDtypeStruct((B, N), jnp.float32),
    )(gamma)


def _pack_idx(idx):
    r = idx.astype(jnp.int32).reshape(-1, 2, 16)
    return (r[:, 0, :] | (r[:, 1, :] << 16)).reshape(-1)


@functools.cache
def _sc_gather(B, N, K):
    NC, NS = 2, 16
    NW = NC * NS
    RPT = B // NW
    OCH = 8192
    GRP = OCH // 32
    KP = K // 2
    assert K == 4 * OCH and B % NW == 0 and K % 32 == 0

    mesh = plsc.VectorSubcoreMesh(core_axis_name="c", subcore_axis_name="s")

    @functools.partial(
        pl.kernel,
        mesh=mesh,
        compiler_params=pltpu.CompilerParams(needs_layout_passes=False),
        out_type=jax.ShapeDtypeStruct((B, K), jnp.float32),
        scratch_types=[
            pltpu.VMEM((KP,), jnp.int32),
            pltpu.VMEM((KP,), jnp.int32),
            pltpu.VMEM((KP,), jnp.int32),
            pltpu.VMEM((N,), jnp.float32),
            pltpu.VMEM((N,), jnp.float32),
            pltpu.VMEM((N,), jnp.float32),
            pltpu.VMEM((OCH,), jnp.float32),
            pltpu.VMEM((OCH,), jnp.float32),
            pltpu.SemaphoreType.DMA,
            pltpu.SemaphoreType.DMA,
            pltpu.SemaphoreType.DMA,
        ],
    )
    def sc(a_hbm, b_hbm, g_hbm, pai_hbm, pbi_hbm, pgi_hbm, out_hbm,
           pai, pbi, pgi, arow, brow, grow, ob0, ob1,
           sem_in, sem_o0, sem_o1):
        wid = lax.axis_index("s") * NC + lax.axis_index("c")
        pltpu.sync_copy(pai_hbm, pai)
        pltpu.sync_copy(pbi_hbm, pbi)
        pltpu.sync_copy(pgi_hbm, pgi)
        row0 = wid * RPT
        m16 = jnp.int32(0xFFFF)

        def do_chunk(c, ob):
            wbase = c * (OCH // 2)

            @plsc.parallel_loop(0, GRP, unroll=4)
            def grp(g):
                w = wbase + g * 16
                wa = pai[pl.ds(w, 16)]
                wb = pbi[pl.ds(w, 16)]
                wg = pgi[pl.ds(w, 16)]
                alo = plsc.load_gather(arow, [lax.bitwise_and(wa, m16)])
                ahi = plsc.load_gather(arow, [lax.shift_right_logical(wa, 16)])
                blo = plsc.load_gather(brow, [lax.bitwise_and(wb, m16)])
                bhi = plsc.load_gather(brow, [lax.shift_right_logical(wb, 16)])
                glo = plsc.load_gather(grow, [lax.bitwise_and(wg, m16)])
                ghi = plsc.load_gather(grow, [lax.shift_right_logical(wg, 16)])
                o = g * 32
                ob[pl.ds(o, 16)] = jnp.maximum(alo + blo - glo, 0.0)
                ob[pl.ds(o + 16, 16)] = jnp.maximum(ahi + bhi - ghi, 0.0)

        def row(r, carry):
            b = row0 + r
            ca = pltpu.async_copy(a_hbm.at[b], arow, sem_in)
            cb = pltpu.async_copy(b_hbm.at[b], brow, sem_in)
            cg = pltpu.async_copy(g_hbm.at[b], grow, sem_in)
            ca.wait()
            cb.wait()
            cg.wait()
            do_chunk(0, ob0)
            o0 = pltpu.async_copy(ob0, out_hbm.at[b, pl.ds(0 * OCH, OCH)], sem_o0)
            do_chunk(1, ob1)
            o1 = pltpu.async_copy(ob1, out_hbm.at[b, pl.ds(1 * OCH, OCH)], sem_o1)
            o0.wait()
            do_chunk(2, ob0)
            o2 = pltpu.async_copy(ob0, out_hbm.at[b, pl.ds(2 * OCH, OCH)], sem_o0)
            o1.wait()
            do_chunk(3, ob1)
            o3 = pltpu.async_copy(ob1, out_hbm.at[b, pl.ds(3 * OCH, OCH)], sem_o1)
            o2.wait()
            o3.wait()
            return carry

        lax.fori_loop(0, RPT, row, 0)

    return sc


def kernel(alpha, beta, gamma, alpha_idx, beta_idx, gamma_idx):
    B, N = alpha.shape
    K = alpha_idx.shape[0]
    ng = _not_gamma(gamma)
    pai = _pack_idx(alpha_idx)
    pbi = _pack_idx(beta_idx)
    pgi = _pack_idx(gamma_idx)
    return _sc_gather(B, N, K)(alpha, beta, ng, pai, pbi, pgi)


# speedup vs baseline: 1.9418x; 1.0020x over previous
import functools

import jax
import jax.numpy as jnp
from jax import lax
from jax.experimental import pallas as pl
from jax.experimental.pallas import tpu as pltpu
from jax.experimental.pallas import tpu_sc as plsc

_VERY_SMALL = 1e-8


def _not_gamma(gamma):
    B, N = gamma.shape
    blk = 64

    def body(g_ref, o_ref):
        g = g_ref[...]
        o_ref[...] = jnp.log(jnp.maximum(1.0 - jnp.exp(g), _VERY_SMALL))

    return pl.pallas_call(
        body,
        grid=(B // blk,),
        in_specs=[pl.BlockSpec((blk, N), lambda i: (i, 0))],
        out_specs=pl.BlockSpec((blk, N), lambda i: (i, 0)),
        out_shape=jax.ShapeDtypeStruct((B, N), jnp.float32),
    )(gamma)


def _pack_idx(idx):
    r = idx.astype(jnp.int32).reshape(-1, 2, 16)
    return (r[:, 0, :] | (r[:, 1, :] << 16)).reshape(-1)


@functools.cache
def _sc_gather(B, N, K):
    NC, NS = 2, 16
    NW = NC * NS
    RPT = B // NW
    OCH = 4096
    NCH = K // OCH
    GRP = OCH // 32
    KP = K // 2
    assert B % NW == 0 and RPT % 2 == 0 and K % OCH == 0 and K % 32 == 0

    mesh = plsc.VectorSubcoreMesh(core_axis_name="c", subcore_axis_name="s")

    @functools.partial(
        pl.kernel,
        mesh=mesh,
        compiler_params=pltpu.CompilerParams(needs_layout_passes=False),
        out_type=jax.ShapeDtypeStruct((B, K), jnp.float32),
        scratch_types=[
            pltpu.VMEM((KP,), jnp.int32),
            pltpu.VMEM((KP,), jnp.int32),
            pltpu.VMEM((KP,), jnp.int32),
            pltpu.VMEM((N,), jnp.float32),
            pltpu.VMEM((N,), jnp.float32),
            pltpu.VMEM((N,), jnp.float32),
            pltpu.VMEM((N,), jnp.float32),
            pltpu.VMEM((OCH,), jnp.float32),
            pltpu.VMEM((OCH,), jnp.float32),
            pltpu.SemaphoreType.DMA,
            pltpu.SemaphoreType.DMA,
            pltpu.SemaphoreType.DMA,
            pltpu.SemaphoreType.DMA,
        ],
    )
    def sc(a_hbm, b_hbm, g_hbm, pai_hbm, pbi_hbm, pgi_hbm, out_hbm,
           pai, pbi, pgi, arow0, arow1, brow, grow, ob0, ob1,
           sem_in, sem_a, sem_o0, sem_o1):
        wid = lax.axis_index("s") * NC + lax.axis_index("c")
        pltpu.sync_copy(pai_hbm, pai)
        pltpu.sync_copy(pbi_hbm, pbi)
        pltpu.sync_copy(pgi_hbm, pgi)
        row0 = wid * RPT
        m16 = jnp.int32(0xFFFF)

        def wait_a(buf):
            pltpu.make_async_copy(a_hbm.at[0], buf, sem_a).wait()

        def wait_bg():
            pltpu.make_async_copy(b_hbm.at[0], brow, sem_in).wait()
            pltpu.make_async_copy(g_hbm.at[0], grow, sem_in).wait()

        def do_chunk(c, ob, atab):
            wbase = c * (OCH // 2)

            @plsc.parallel_loop(0, GRP, unroll=4)
            def grp(g):
                w = wbase + g * 16
                wa = pai[pl.ds(w, 16)]
                wb = pbi[pl.ds(w, 16)]
                wg = pgi[pl.ds(w, 16)]
                alo = plsc.load_gather(atab, [lax.bitwise_and(wa, m16)])
                ahi = plsc.load_gather(atab, [lax.shift_right_logical(wa, 16)])
                blo = plsc.load_gather(brow, [lax.bitwise_and(wb, m16)])
                bhi = plsc.load_gather(brow, [lax.shift_right_logical(wb, 16)])
                glo = plsc.load_gather(grow, [lax.bitwise_and(wg, m16)])
                ghi = plsc.load_gather(grow, [lax.shift_right_logical(wg, 16)])
                o = g * 32
                ob[pl.ds(o, 16)] = jnp.maximum(alo + blo - glo, 0.0)
                ob[pl.ds(o + 16, 16)] = jnp.maximum(ahi + bhi - ghi, 0.0)

        def do_row(b, atab, pending):
            for c in range(NCH):
                ob, slot, sem = (ob0, 0, sem_o0) if c % 2 == 0 else (ob1, 1, sem_o1)
                if pending[slot] is not None:
                    pending[slot].wait()
                do_chunk(c, ob, atab)
                pending[slot] = pltpu.async_copy(
                    ob, out_hbm.at[b, pl.ds(c * OCH, OCH)], sem)

        pltpu.async_copy(a_hbm.at[row0], arow0, sem_a)
        pltpu.async_copy(b_hbm.at[row0], brow, sem_in)
        pltpu.async_copy(g_hbm.at[row0], grow, sem_in)

        def pair(i, carry):
            r0 = row0 + 2 * i
            pending = [None, None]
            pltpu.async_copy(a_hbm.at[r0 + 1], arow1, sem_a)
            wait_a(arow0)
            wait_bg()
            do_row(r0, arow0, pending)
            nxt = jnp.minimum(r0 + 2, row0 + RPT - 1)
            pltpu.async_copy(a_hbm.at[nxt], arow0, sem_a)
            pltpu.async_copy(b_hbm.at[r0 + 1], brow, sem_in)
            pltpu.async_copy(g_hbm.at[r0 + 1], grow, sem_in)
            wait_a(arow1)
            wait_bg()
            do_row(r0 + 1, arow1, pending)
            pltpu.async_copy(b_hbm.at[nxt], brow, sem_in)
            pltpu.async_copy(g_hbm.at[nxt], grow, sem_in)
            pending[0].wait()
            pending[1].wait()
            return carry

        lax.fori_loop(0, RPT // 2, pair, 0)
        wait_a(arow0)
        wait_bg()

    return sc


def kernel(alpha, beta, gamma, alpha_idx, beta_idx, gamma_idx):
    B, N = alpha.shape
    K = alpha_idx.shape[0]
    ng = _not_gamma(gamma)
    pai = _pack_idx(alpha_idx)
    pbi = _pack_idx(beta_idx)
    pgi = _pack_idx(gamma_idx)
    return _sc_gather(B, N, K)(alpha, beta, ng, pai, pbi, pgi)
